# baseline (speedup 1.0000x reference)
import dataclasses
import functools

import jax
import jax.numpy as jnp
from jax import lax
from jax.experimental import pallas as pl
from jax.experimental.pallas import tpu as pltpu
from jax.experimental.pallas import tpu_sc as plsc

_K = 4
_ROWS_PER_BLOCK = 1024
_GATHER_WINDOW = 128


def _dist_topk_body(x_ref, e_ref, idx_ref, diff_ref, cbt_ref, qst_ref, *,
                    denom):
    pid = pl.program_id(0)
    nprog = pl.num_programs(0)
    x = x_ref[...]
    e = e_ref[...]
    mm = lax.dot_general(x, e, (((1,), (0,)), ((), ())),
                         preferred_element_type=jnp.float32)
    x2 = jnp.sum(x * x, axis=1, keepdims=True)
    e2 = jnp.sum(e * e, axis=0, keepdims=True)
    dist = (x2 - 2.0 * mm) + e2
    colf = lax.broadcasted_iota(jnp.int32, dist.shape, 1).astype(jnp.float32)
    inf = jnp.float32(jnp.inf)
    idx_cols = []
    bsum = jnp.float32(0.0)
    onehot = None
    for k in range(_K):
        m = jnp.min(dist, axis=1, keepdims=True)
        idxf = jnp.min(jnp.where(dist == m, colf, inf), axis=1)
        idx_cols.append(idxf.astype(jnp.int32))
        if k == 0:
            bsum = jnp.sum(m)
            onehot = (colf == idxf[:, None]).astype(jnp.bfloat16)
        if k < _K - 1:
            dist = jnp.where(colf == idxf[:, None], inf, dist)
    idx_ref[...] = jnp.stack(idx_cols, axis=1)
    e_hi = e.astype(jnp.bfloat16)
    e_lo = (e - e_hi.astype(jnp.float32)).astype(jnp.bfloat16)
    q = (lax.dot_general(onehot, e_hi, (((1,), (1,)), ((), ())),
                         preferred_element_type=jnp.float32)
         + lax.dot_general(onehot, e_lo, (((1,), (1,)), ((), ())),
                           preferred_element_type=jnp.float32))
    qst_ref[...] = x + (q - x)
    acc = jnp.where(pid == 0, 0.0, diff_ref[0, 0]) + bsum
    diff_ref[0, 0] = jnp.where(pid == nprog - 1, acc / denom, acc)

    @pl.when(pid == 0)
    def _():
        cbt_ref[...] = e.T


def _dist_topk(flat, embed):
    n_rows, dim = flat.shape
    n_embed = embed.shape[1]
    r = _ROWS_PER_BLOCK
    grid = n_rows // r
    return pl.pallas_call(
        functools.partial(_dist_topk_body, denom=float(n_rows * dim)),
        grid=(grid,),
        in_specs=[
            pl.BlockSpec((r, dim), lambda i: (i, 0)),
            pl.BlockSpec((dim, n_embed), lambda i: (0, 0)),
        ],
        out_specs=[
            pl.BlockSpec((r, _K), lambda i: (i, 0)),
            pl.BlockSpec((1, 1), lambda i: (0, 0), memory_space=pltpu.SMEM),
            pl.BlockSpec((n_embed, dim), lambda i: (0, 0)),
            pl.BlockSpec((r, dim), lambda i: (i, 0)),
        ],
        out_shape=[
            jax.ShapeDtypeStruct((n_rows, _K), jnp.int32),
            jax.ShapeDtypeStruct((1, 1), jnp.float32),
            jax.ShapeDtypeStruct((n_embed, dim), jnp.float32),
            jax.ShapeDtypeStruct((n_rows, dim), jnp.float32),
        ],
    )(flat, embed)


def _sc_gather(cbt, idx_flat):
    n = idx_flat.shape[1]
    n_embed, dim = cbt.shape
    gw = _GATHER_WINDOW
    lanes = 16
    mesh = plsc.VectorSubcoreMesh(core_axis_name="core",
                                  subcore_axis_name="subcore")
    cp = pltpu.CompilerParams(use_tc_tiling_on_sc=False)
    if "needs_layout_passes" in pltpu.CompilerParams.__dataclass_fields__:
        cp = dataclasses.replace(cp, needs_layout_passes=False)

    @functools.partial(
        pl.kernel,
        out_type=jax.ShapeDtypeStruct((n, dim), jnp.float32),
        mesh=mesh,
        scratch_types=[pltpu.VMEM((n_embed, dim), jnp.float32)],
        compiler_params=cp,
    )
    def gk(cbt_hbm, i_hbm, o_hbm, table_v):
        pltpu.sync_copy(cbt_hbm, table_v)
        lane_iota = lax.iota(jnp.int32, lanes)

        def body(i_vmem, o_vmem):
            for g in range(gw // lanes):
                rows = i_vmem[0, pl.ds(g * lanes, lanes)]
                out_rows = g * lanes + lane_iota
                for j in range(dim):
                    cj = jnp.full((lanes,), j, jnp.int32)
                    v = plsc.load_gather(table_v, [rows, cj])
                    plsc.store_scatter(o_vmem, [out_rows, cj], v)

        pltpu.emit_pipeline(
            body,
            grid=(n // gw,),
            in_specs=[pl.BlockSpec((1, gw), lambda i: (0, i))],
            out_specs=[pl.BlockSpec((gw, dim), lambda i: (i, 0))],
            core_axis_name=("core", "subcore"),
            dimension_semantics=(pltpu.PARALLEL,),
        )(i_hbm, o_hbm)

    return gk(cbt, idx_flat)


def kernel(input, embed):
    b, h, w, dim = input.shape
    flat = input.reshape(-1, dim)
    n_rows = flat.shape[0]
    idx, diff_acc, cbt, qst = _dist_topk(flat, embed)
    gathered = _sc_gather(cbt, idx.reshape(1, n_rows * _K))
    quantize_topk = gathered.reshape(b, h, w, _K * dim)
    diff = diff_acc[0, 0]
    quantize_st = qst.reshape(b, h, w, dim)
    return (quantize_topk, diff, quantize_st)


# speedup vs baseline: 1.5362x; 1.5362x over previous
import dataclasses
import functools

import jax
import jax.numpy as jnp
from jax import lax
from jax.experimental import pallas as pl
from jax.experimental.pallas import tpu as pltpu
from jax.experimental.pallas import tpu_sc as plsc

_K = 4
_ROWS_PER_BLOCK = 1024
_CHUNKS = 4
_GATHER_WINDOW = 128


def _dist_topk_body(x_ref, e_ref, idx_ref, diff_ref, cbt_ref, qst_ref):
    pid = pl.program_id(0)
    x = x_ref[...]
    e = e_ref[...]
    mm = lax.dot_general(x, e, (((1,), (0,)), ((), ())),
                         preferred_element_type=jnp.float32)
    x2 = jnp.sum(x * x, axis=1, keepdims=True)
    e2 = jnp.sum(e * e, axis=0, keepdims=True)
    dist = (x2 - 2.0 * mm) + e2
    colf = lax.broadcasted_iota(jnp.int32, dist.shape, 1).astype(jnp.float32)
    inf = jnp.float32(jnp.inf)
    idx_cols = []
    bsum = jnp.float32(0.0)
    onehot = None
    for k in range(_K):
        m = jnp.min(dist, axis=1, keepdims=True)
        idxf = jnp.min(jnp.where(dist == m, colf, inf), axis=1)
        idx_cols.append(idxf.astype(jnp.int32))
        if k == 0:
            bsum = jnp.sum(m)
            onehot = (colf == idxf[:, None]).astype(jnp.bfloat16)
        if k < _K - 1:
            dist = jnp.where(colf == idxf[:, None], inf, dist)
    idx_ref[...] = jnp.stack(idx_cols, axis=1)
    e_hi = e.astype(jnp.bfloat16)
    e_lo = (e - e_hi.astype(jnp.float32)).astype(jnp.bfloat16)
    q = (lax.dot_general(onehot, e_hi, (((1,), (1,)), ((), ())),
                         preferred_element_type=jnp.float32)
         + lax.dot_general(onehot, e_lo, (((1,), (1,)), ((), ())),
                           preferred_element_type=jnp.float32))
    qst_ref[...] = x + (q - x)
    diff_ref[0, 0] = jnp.where(pid == 0, 0.0, diff_ref[0, 0]) + bsum

    @pl.when(pid == 0)
    def _():
        cbt_ref[...] = e.T


def _dist_topk(flat, embed):
    n_rows, dim = flat.shape
    n_embed = embed.shape[1]
    r = _ROWS_PER_BLOCK
    grid = n_rows // r
    return pl.pallas_call(
        _dist_topk_body,
        grid=(grid,),
        in_specs=[
            pl.BlockSpec((r, dim), lambda i: (i, 0)),
            pl.BlockSpec((dim, n_embed), lambda i: (0, 0)),
        ],
        out_specs=[
            pl.BlockSpec((r, _K), lambda i: (i, 0)),
            pl.BlockSpec((1, 1), lambda i: (0, 0), memory_space=pltpu.SMEM),
            pl.BlockSpec((n_embed, dim), lambda i: (0, 0)),
            pl.BlockSpec((r, dim), lambda i: (i, 0)),
        ],
        out_shape=[
            jax.ShapeDtypeStruct((n_rows, _K), jnp.int32),
            jax.ShapeDtypeStruct((1, 1), jnp.float32),
            jax.ShapeDtypeStruct((n_embed, dim), jnp.float32),
            jax.ShapeDtypeStruct((n_rows, dim), jnp.float32),
        ],
    )(flat, embed)


def _sc_gather(cbt, idx_flat):
    n = idx_flat.shape[1]
    dim = cbt.shape[1]
    gw = _GATHER_WINDOW
    mesh = plsc.VectorSubcoreMesh(core_axis_name="core",
                                  subcore_axis_name="subcore")

    @functools.partial(
        pl.kernel,
        out_type=jax.ShapeDtypeStruct((n, dim), jnp.float32),
        mesh=mesh,
        compiler_params=pltpu.CompilerParams(use_tc_tiling_on_sc=False),
    )
    def gk(cbt_hbm, i_hbm, o_hbm):
        def body(i_vmem, o_vmem):
            pltpu.sync_copy(cbt_hbm.at[i_vmem.at[0]], o_vmem)

        pltpu.emit_pipeline(
            body,
            grid=(n // gw,),
            in_specs=[pl.BlockSpec((1, gw), lambda i: (0, i))],
            out_specs=[pl.BlockSpec((gw, dim), lambda i: (i, 0))],
            core_axis_name=("core", "subcore"),
            dimension_semantics=(pltpu.PARALLEL,),
        )(i_hbm, o_hbm)

    return gk(cbt, idx_flat)


def kernel(input, embed):
    b, h, w, dim = input.shape
    nb = b // _CHUNKS
    cbt = None
    qtopk_parts, qst_parts, dsums = [], [], []
    for c in range(_CHUNKS):
        flat_c = input[c * nb:(c + 1) * nb].reshape(-1, dim)
        idx, dsum, cbt_c, qst = _dist_topk(flat_c, embed)
        if cbt is None:
            cbt = cbt_c
        g = _sc_gather(cbt, idx.reshape(1, flat_c.shape[0] * _K))
        qtopk_parts.append(g.reshape(nb, h, w, _K * dim))
        qst_parts.append(qst.reshape(nb, h, w, dim))
        dsums.append(dsum[0, 0])
    quantize_topk = jnp.concatenate(qtopk_parts, axis=0)
    quantize_st = jnp.concatenate(qst_parts, axis=0)
    diff = sum(dsums) / float(b * h * w * dim)
    return (quantize_topk, diff, quantize_st)


# speedup vs baseline: 1.5748x; 1.0251x over previous
import dataclasses
import functools

import jax
import numpy as np
import jax.numpy as jnp
from jax import lax
from jax.experimental import pallas as pl
from jax.experimental.pallas import tpu as pltpu
from jax.experimental.pallas import tpu_sc as plsc

_K = 4
_ROWS_PER_BLOCK = 1024
_CHUNKS = 4
_GATHER_WINDOW = 128


def _dist_topk_body(x_ref, e_ref, idx_ref, diff_ref, cbt_ref, qst_ref):
    pid = pl.program_id(0)
    x = x_ref[...]
    e = e_ref[...]
    mm = lax.dot_general(x, e, (((1,), (0,)), ((), ())),
                         preferred_element_type=jnp.float32)
    x2 = jnp.sum(x * x, axis=1, keepdims=True)
    e2 = jnp.sum(e * e, axis=0, keepdims=True)
    dist = (x2 - 2.0 * mm) + e2
    colf = lax.broadcasted_iota(jnp.int32, dist.shape, 1).astype(jnp.float32)
    inf = jnp.float32(jnp.inf)
    idx_cols = []
    bsum = jnp.float32(0.0)
    onehot = None
    for k in range(_K):
        m = jnp.min(dist, axis=1, keepdims=True)
        idxf = jnp.min(jnp.where(dist == m, colf, inf), axis=1)
        idx_cols.append(idxf.astype(jnp.int32))
        if k == 0:
            bsum = jnp.sum(m)
            onehot = (colf == idxf[:, None]).astype(jnp.bfloat16)
        if k < _K - 1:
            dist = jnp.where(colf == idxf[:, None], inf, dist)
    idx_ref[...] = jnp.stack(idx_cols, axis=1)
    e_hi = e.astype(jnp.bfloat16)
    e_lo = (e - e_hi.astype(jnp.float32)).astype(jnp.bfloat16)
    q = (lax.dot_general(onehot, e_hi, (((1,), (1,)), ((), ())),
                         preferred_element_type=jnp.float32)
         + lax.dot_general(onehot, e_lo, (((1,), (1,)), ((), ())),
                           preferred_element_type=jnp.float32))
    qst_ref[...] = x + (q - x)
    diff_ref[0, 0] = jnp.where(pid == 0, 0.0, diff_ref[0, 0]) + bsum

    @pl.when(pid == 0)
    def _():
        cbt_ref[...] = e.T


def _dist_topk(flat, embed):
    n_rows, dim = flat.shape
    n_embed = embed.shape[1]
    r = _ROWS_PER_BLOCK
    grid = n_rows // r
    return pl.pallas_call(
        _dist_topk_body,
        grid=(grid,),
        in_specs=[
            pl.BlockSpec((r, dim), lambda i: (i, 0)),
            pl.BlockSpec((dim, n_embed), lambda i: (0, 0)),
        ],
        out_specs=[
            pl.BlockSpec((r, _K), lambda i: (i, 0)),
            pl.BlockSpec((1, 1), lambda i: (0, 0), memory_space=pltpu.SMEM),
            pl.BlockSpec((n_embed, dim), lambda i: (0, 0)),
            pl.BlockSpec((r, dim), lambda i: (i, 0)),
        ],
        out_shape=[
            jax.ShapeDtypeStruct((n_rows, _K), jnp.int32),
            jax.ShapeDtypeStruct((1, 1), jnp.float32),
            jax.ShapeDtypeStruct((n_embed, dim), jnp.float32),
            jax.ShapeDtypeStruct((n_rows, dim), jnp.float32),
        ],
    )(flat, embed)


def _sc_gather(cbt, idx_flat):
    n = idx_flat.shape[1]
    n_embed, dim = cbt.shape
    gw = _GATHER_WINDOW
    lanes = 16
    mesh = plsc.VectorSubcoreMesh(core_axis_name="core",
                                  subcore_axis_name="subcore")
    cp = pltpu.CompilerParams(use_tc_tiling_on_sc=False)
    if "needs_layout_passes" in pltpu.CompilerParams.__dataclass_fields__:
        cp = dataclasses.replace(cp, needs_layout_passes=False)
    @functools.partial(
        pl.kernel,
        out_type=jax.ShapeDtypeStruct((n, dim), jnp.float32),
        mesh=mesh,
        scratch_types=[pltpu.VMEM((n_embed, dim), jnp.float32)],
        compiler_params=cp,
    )
    def gk(cbt_hbm, i_hbm, o_hbm, table_v):
        pltpu.sync_copy(cbt_hbm, table_v)

        def body(i_vmem, o_vmem):
            iota16 = lax.iota(jnp.int32, lanes)
            for g in range(gw // lanes):
                codes = i_vmem[0, pl.ds(g * lanes, lanes)]
                orow = iota16 + (g * lanes)
                for j in range(dim):
                    cv = (iota16 + j) & (dim - 1)
                    v = plsc.load_gather(table_v, [codes, cv])
                    plsc.store_scatter(o_vmem, [orow, cv], v)

        pltpu.emit_pipeline(
            body,
            grid=(n // gw,),
            in_specs=[pl.BlockSpec((1, gw), lambda i: (0, i))],
            out_specs=[pl.BlockSpec((gw, dim), lambda i: (i, 0))],
            core_axis_name=("core", "subcore"),
            dimension_semantics=(pltpu.PARALLEL,),
        )(i_hbm, o_hbm)

    return gk(cbt, idx_flat)


def kernel(input, embed):
    b, h, w, dim = input.shape
    nb = b // _CHUNKS
    cbt = None
    qtopk_parts, qst_parts, dsums = [], [], []
    for c in range(_CHUNKS):
        flat_c = input[c * nb:(c + 1) * nb].reshape(-1, dim)
        idx, dsum, cbt_c, qst = _dist_topk(flat_c, embed)
        if cbt is None:
            cbt = cbt_c
        g = _sc_gather(cbt, idx.reshape(1, flat_c.shape[0] * _K))
        qtopk_parts.append(g.reshape(nb, h, w, _K * dim))
        qst_parts.append(qst.reshape(nb, h, w, dim))
        dsums.append(dsum[0, 0])
    quantize_topk = jnp.concatenate(qtopk_parts, axis=0)
    quantize_st = jnp.concatenate(qst_parts, axis=0)
    diff = sum(dsums) / float(b * h * w * dim)
    return (quantize_topk, diff, quantize_st)


# speedup vs baseline: 1.6622x; 1.0555x over previous
import dataclasses
import functools

import jax
import numpy as np
import jax.numpy as jnp
from jax import lax
from jax.experimental import pallas as pl
from jax.experimental.pallas import tpu as pltpu
from jax.experimental.pallas import tpu_sc as plsc

_K = 4
_ROWS_PER_BLOCK = 1024
_CHUNKS = 4
_GATHER_WINDOW = 192


def _dist_topk_body(x_ref, e_ref, idx_ref, diff_ref, cbt_ref, qst_ref):
    pid = pl.program_id(0)
    x = x_ref[...]
    e = e_ref[...]
    mm = lax.dot_general(x, e, (((1,), (0,)), ((), ())),
                         preferred_element_type=jnp.float32)
    x2 = jnp.sum(x * x, axis=1, keepdims=True)
    e2 = jnp.sum(e * e, axis=0, keepdims=True)
    dist = (x2 - 2.0 * mm) + e2
    colf = lax.broadcasted_iota(jnp.int32, dist.shape, 1).astype(jnp.float32)
    inf = jnp.float32(jnp.inf)
    idx_cols = []
    bsum = jnp.float32(0.0)
    onehot = None
    for k in range(_K):
        m = jnp.min(dist, axis=1, keepdims=True)
        idxf = jnp.min(jnp.where(dist == m, colf, inf), axis=1)
        idx_cols.append(idxf.astype(jnp.int32))
        if k == 0:
            bsum = jnp.sum(m)
            onehot = (colf == idxf[:, None]).astype(jnp.bfloat16)
        if k < _K - 1:
            dist = jnp.where(colf == idxf[:, None], inf, dist)
    idx_ref[...] = jnp.stack(idx_cols, axis=1)
    e_hi = e.astype(jnp.bfloat16)
    e_lo = (e - e_hi.astype(jnp.float32)).astype(jnp.bfloat16)
    q = (lax.dot_general(onehot, e_hi, (((1,), (1,)), ((), ())),
                         preferred_element_type=jnp.float32)
         + lax.dot_general(onehot, e_lo, (((1,), (1,)), ((), ())),
                           preferred_element_type=jnp.float32))
    qst_ref[...] = x + (q - x)
    diff_ref[0, 0] = jnp.where(pid == 0, 0.0, diff_ref[0, 0]) + bsum

    @pl.when(pid == 0)
    def _():
        cbt_ref[...] = e.T


def _dist_topk(flat, embed):
    n_rows, dim = flat.shape
    n_embed = embed.shape[1]
    r = _ROWS_PER_BLOCK
    grid = n_rows // r
    return pl.pallas_call(
        _dist_topk_body,
        grid=(grid,),
        in_specs=[
            pl.BlockSpec((r, dim), lambda i: (i, 0)),
            pl.BlockSpec((dim, n_embed), lambda i: (0, 0)),
        ],
        out_specs=[
            pl.BlockSpec((r, _K), lambda i: (i, 0)),
            pl.BlockSpec((1, 1), lambda i: (0, 0), memory_space=pltpu.SMEM),
            pl.BlockSpec((n_embed, dim), lambda i: (0, 0)),
            pl.BlockSpec((r, dim), lambda i: (i, 0)),
        ],
        out_shape=[
            jax.ShapeDtypeStruct((n_rows, _K), jnp.int32),
            jax.ShapeDtypeStruct((1, 1), jnp.float32),
            jax.ShapeDtypeStruct((n_embed, dim), jnp.float32),
            jax.ShapeDtypeStruct((n_rows, dim), jnp.float32),
        ],
    )(flat, embed)


def _sc_gather(cbt, idx_flat):
    n = idx_flat.shape[1]
    n_embed, dim = cbt.shape
    gw = _GATHER_WINDOW
    lanes = 16
    mesh = plsc.VectorSubcoreMesh(core_axis_name="core",
                                  subcore_axis_name="subcore")
    cp = pltpu.CompilerParams(use_tc_tiling_on_sc=False)
    if "needs_layout_passes" in pltpu.CompilerParams.__dataclass_fields__:
        cp = dataclasses.replace(cp, needs_layout_passes=False)
    @functools.partial(
        pl.kernel,
        out_type=jax.ShapeDtypeStruct((n, dim), jnp.float32),
        mesh=mesh,
        scratch_types=[pltpu.VMEM((n_embed, dim), jnp.float32)],
        compiler_params=cp,
    )
    def gk(cbt_hbm, i_hbm, o_hbm, table_v):
        pltpu.sync_copy(cbt_hbm, table_v)

        def body(i_vmem, o_vmem):
            iota16 = lax.iota(jnp.int32, lanes)
            for g in range(gw // lanes):
                codes = i_vmem[0, pl.ds(g * lanes, lanes)]
                orow = iota16 + (g * lanes)
                for j in range(dim):
                    cv = (iota16 + j) & (dim - 1)
                    v = plsc.load_gather(table_v, [codes, cv])
                    plsc.store_scatter(o_vmem, [orow, cv], v)

        pltpu.emit_pipeline(
            body,
            grid=(n // gw,),
            in_specs=[pl.BlockSpec((1, gw), lambda i: (0, i))],
            out_specs=[pl.BlockSpec((gw, dim), lambda i: (i, 0))],
            core_axis_name=("core", "subcore"),
            dimension_semantics=(pltpu.PARALLEL,),
        )(i_hbm, o_hbm)

    return gk(cbt, idx_flat)


def kernel(input, embed):
    b, h, w, dim = input.shape
    nb = b // _CHUNKS
    cbt = None
    qtopk_parts, qst_parts, dsums = [], [], []
    for c in range(_CHUNKS):
        flat_c = input[c * nb:(c + 1) * nb].reshape(-1, dim)
        idx, dsum, cbt_c, qst = _dist_topk(flat_c, embed)
        if cbt is None:
            cbt = cbt_c
        g = _sc_gather(cbt, idx.reshape(1, flat_c.shape[0] * _K))
        qtopk_parts.append(g.reshape(nb, h, w, _K * dim))
        qst_parts.append(qst.reshape(nb, h, w, dim))
        dsums.append(dsum[0, 0])
    quantize_topk = jnp.zeros((b, h, w, _K * dim), jnp.float32)
    quantize_st = jnp.zeros((b, h, w, dim), jnp.float32)
    for c in range(_CHUNKS):
        quantize_topk = lax.dynamic_update_slice(
            quantize_topk, qtopk_parts[c], (c * nb, 0, 0, 0))
        quantize_st = lax.dynamic_update_slice(
            quantize_st, qst_parts[c], (c * nb, 0, 0, 0))
    diff = sum(dsums) / float(b * h * w * dim)
    return (quantize_topk, diff, quantize_st)


# speedup vs baseline: 1.7409x; 1.0474x over previous
import dataclasses
import functools

import jax
import numpy as np
import jax.numpy as jnp
from jax import lax
from jax.experimental import pallas as pl
from jax.experimental.pallas import tpu as pltpu
from jax.experimental.pallas import tpu_sc as plsc

_K = 4
_ROWS_PER_BLOCK = 1024
_CHUNKS = 4
_GATHER_WINDOW = 192


def _dist_topk_body(x_ref, e_ref, idx_ref, diff_ref, cbt_ref, qst_ref):
    pid = pl.program_id(0)
    x = x_ref[...]
    e = e_ref[...]
    mm = lax.dot_general(x, e, (((1,), (0,)), ((), ())),
                         preferred_element_type=jnp.float32)
    x2 = jnp.sum(x * x, axis=1, keepdims=True)
    e2 = jnp.sum(e * e, axis=0, keepdims=True)
    dist = (x2 - 2.0 * mm) + e2
    colf = lax.broadcasted_iota(jnp.int32, dist.shape, 1).astype(jnp.float32)
    inf = jnp.float32(jnp.inf)
    idx_cols = []
    bsum = jnp.float32(0.0)
    onehot = None
    for k in range(_K):
        m = jnp.min(dist, axis=1, keepdims=True)
        idxf = jnp.min(jnp.where(dist == m, colf, inf), axis=1)
        idx_cols.append(idxf.astype(jnp.int32))
        if k == 0:
            bsum = jnp.sum(m)
            onehot = (colf == idxf[:, None]).astype(jnp.bfloat16)
        if k < _K - 1:
            dist = jnp.where(colf == idxf[:, None], inf, dist)
    idx_ref[...] = jnp.stack(idx_cols, axis=1)
    e_hi = e.astype(jnp.bfloat16)
    e_lo = (e - e_hi.astype(jnp.float32)).astype(jnp.bfloat16)
    q = (lax.dot_general(onehot, e_hi, (((1,), (1,)), ((), ())),
                         preferred_element_type=jnp.float32)
         + lax.dot_general(onehot, e_lo, (((1,), (1,)), ((), ())),
                           preferred_element_type=jnp.float32))
    qst_ref[...] = x + (q - x)
    diff_ref[0, 0] = jnp.where(pid == 0, 0.0, diff_ref[0, 0]) + bsum

    @pl.when(pid == 0)
    def _():
        cbt_ref[...] = e.T


def _dist_topk(flat, embed):
    n_rows, dim = flat.shape
    n_embed = embed.shape[1]
    r = _ROWS_PER_BLOCK
    grid = n_rows // r
    return pl.pallas_call(
        _dist_topk_body,
        grid=(grid,),
        in_specs=[
            pl.BlockSpec((r, dim), lambda i: (i, 0)),
            pl.BlockSpec((dim, n_embed), lambda i: (0, 0)),
        ],
        out_specs=[
            pl.BlockSpec((r, _K), lambda i: (i, 0)),
            pl.BlockSpec((1, 1), lambda i: (0, 0), memory_space=pltpu.SMEM),
            pl.BlockSpec((n_embed, dim), lambda i: (0, 0)),
            pl.BlockSpec((r, dim), lambda i: (i, 0)),
        ],
        out_shape=[
            jax.ShapeDtypeStruct((n_rows, _K), jnp.int32),
            jax.ShapeDtypeStruct((1, 1), jnp.float32),
            jax.ShapeDtypeStruct((n_embed, dim), jnp.float32),
            jax.ShapeDtypeStruct((n_rows, dim), jnp.float32),
        ],
    )(flat, embed)


def _sc_gather(cbt, idx_flat):
    n = idx_flat.shape[1]
    n_embed, dim = cbt.shape
    gw = _GATHER_WINDOW
    lanes = 16
    mesh = plsc.VectorSubcoreMesh(core_axis_name="core",
                                  subcore_axis_name="subcore")
    cp = pltpu.CompilerParams(use_tc_tiling_on_sc=False)
    if "needs_layout_passes" in pltpu.CompilerParams.__dataclass_fields__:
        cp = dataclasses.replace(cp, needs_layout_passes=False)
    @functools.partial(
        pl.kernel,
        out_type=jax.ShapeDtypeStruct((n, dim), jnp.float32),
        mesh=mesh,
        scratch_types=[pltpu.VMEM((n_embed, dim), jnp.float32)],
        compiler_params=cp,
    )
    def gk(cbt_hbm, i_hbm, o_hbm, table_v):
        pltpu.sync_copy(cbt_hbm, table_v)

        def body(i_vmem, o_vmem):
            iota16 = lax.iota(jnp.int32, lanes)
            for g in range(gw // lanes):
                codes = i_vmem[0, pl.ds(g * lanes, lanes)]
                orow = iota16 + (g * lanes)
                for j0 in range(0, dim, 8):
                    vs = []
                    for j in range(j0, j0 + 8):
                        cv = (iota16 + j) & (dim - 1)
                        vs.append((cv, plsc.load_gather(table_v, [codes, cv])))
                    for cv, v in vs:
                        plsc.store_scatter(o_vmem, [orow, cv], v)

        pltpu.emit_pipeline(
            body,
            grid=(n // gw,),
            in_specs=[pl.BlockSpec((1, gw), lambda i: (0, i))],
            out_specs=[pl.BlockSpec((gw, dim), lambda i: (i, 0))],
            core_axis_name=("core", "subcore"),
            dimension_semantics=(pltpu.PARALLEL,),
        )(i_hbm, o_hbm)

    return gk(cbt, idx_flat)


def kernel(input, embed):
    b, h, w, dim = input.shape
    nb = b // _CHUNKS
    cbt = None
    qtopk_parts, qst_parts, dsums = [], [], []
    for c in range(_CHUNKS):
        flat_c = input[c * nb:(c + 1) * nb].reshape(-1, dim)
        idx, dsum, cbt_c, qst = _dist_topk(flat_c, embed)
        if cbt is None:
            cbt = cbt_c
        g = _sc_gather(cbt, idx.reshape(1, flat_c.shape[0] * _K))
        qtopk_parts.append(g.reshape(nb, h, w, _K * dim))
        qst_parts.append(qst.reshape(nb, h, w, dim))
        dsums.append(dsum[0, 0])
    quantize_topk = jnp.zeros((b, h, w, _K * dim), jnp.float32)
    quantize_st = jnp.zeros((b, h, w, dim), jnp.float32)
    for c in range(_CHUNKS):
        quantize_topk = lax.dynamic_update_slice(
            quantize_topk, qtopk_parts[c], (c * nb, 0, 0, 0))
        quantize_st = lax.dynamic_update_slice(
            quantize_st, qst_parts[c], (c * nb, 0, 0, 0))
    diff = sum(dsums) / float(b * h * w * dim)
    return (quantize_topk, diff, quantize_st)
